# Initial kernel scaffold; baseline (speedup 1.0000x reference)
import jax
import jax.numpy as jnp
from jax import lax
from jax.experimental import pallas as pl
from jax.experimental.pallas import tpu as pltpu
from jax.experimental.pallas import tpu_sc as plsc

N_USERS = 25000
M_ITEMS = 25000
N = N_USERS + M_ITEMS
D = 64
DH = D // 2
LAYERS = 3
E = 800000

NC = 2
NT = 16
B = 128
E_TILE = E // NT
NBLK = -(-E_TILE // B)
EP = NBLK * B
NP = 50048
ROWS_TILE = NP // NT


def _sc_mesh():
    return plsc.VectorSubcoreMesh(
        core_axis_name="c", subcore_axis_name="s", num_cores=NC, num_subcores=NT
    )


def _deg_body(idx2, z1, deg_out, degacc, idx_v, ones_v):
    c = lax.axis_index("c")
    s = lax.axis_index("s")
    for i in range(B // 16):
        ones_v[pl.ds(i * 16, 16)] = jnp.full((16,), 1.0, jnp.float32)
    r0 = s * ROWS_TILE
    pltpu.sync_copy(z1.at[pl.ds(r0, ROWS_TILE)], degacc.at[pl.ds(r0, ROWS_TILE)])
    plsc.subcore_barrier()

    def blk(i, carry):
        pltpu.sync_copy(idx2.at[c, s, pl.ds(i * B, B)], idx_v)
        pltpu.sync_copy(ones_v, degacc.at[idx_v], add=True)
        return carry

    lax.fori_loop(0, NBLK, blk, 0)
    plsc.subcore_barrier()
    pltpu.sync_copy(degacc.at[pl.ds(r0, ROWS_TILE)], deg_out.at[c, pl.ds(r0, ROWS_TILE)])


def _deg_call(idx2, z1):
    return pl.kernel(
        _deg_body,
        out_type=jax.ShapeDtypeStruct((NC, NP), jnp.float32),
        mesh=_sc_mesh(),
        scratch_types=[
            pltpu.VMEM_SHARED((NP,), jnp.float32),
            pltpu.VMEM((B,), jnp.int32),
            pltpu.VMEM((B,), jnp.float32),
        ],
    )(idx2, z1)


def _layer_body(idx2, v, z2, seg_out, acc, src_v, dst_v, rows, sem):
    c = lax.axis_index("c")
    s = lax.axis_index("s")
    r0 = s * ROWS_TILE
    pltpu.sync_copy(z2.at[pl.ds(r0, ROWS_TILE)], acc.at[pl.ds(r0, ROWS_TILE)])
    plsc.subcore_barrier()

    def blk(i, carry):
        pltpu.sync_copy(idx2.at[0, s, pl.ds(i * B, B)], src_v)
        pltpu.sync_copy(idx2.at[1, s, pl.ds(i * B, B)], dst_v)
        pltpu.async_copy(v.at[c].at[src_v], rows, sem).wait()
        pltpu.sync_copy(rows, acc.at[dst_v], add=True)
        return carry

    lax.fori_loop(0, NBLK, blk, 0)
    plsc.subcore_barrier()
    pltpu.sync_copy(acc.at[pl.ds(r0, ROWS_TILE)], seg_out.at[c, pl.ds(r0, ROWS_TILE)])


def _layer_call(idx2, v, z2):
    return pl.kernel(
        _layer_body,
        out_type=jax.ShapeDtypeStruct((NC, NP, DH), jnp.float32),
        mesh=_sc_mesh(),
        scratch_types=[
            pltpu.VMEM_SHARED((NP, DH), jnp.float32),
            pltpu.VMEM((B,), jnp.int32),
            pltpu.VMEM((B,), jnp.int32),
            pltpu.VMEM((B, DH), jnp.float32),
            pltpu.SemaphoreType.DMA,
        ],
    )(idx2, v, z2)


_RS = 512


def _setup_body(ds_ref, dd_ref, emb_ref, bexp_ref, cexp_ref, v0_ref):
    a = lax.rsqrt(jnp.maximum(ds_ref[...], 1.0))
    b = lax.rsqrt(jnp.maximum(dd_ref[...], 1.0))
    bexp_ref[...] = jnp.broadcast_to(b, (_RS, DH))
    cexp_ref[...] = jnp.broadcast_to(a * b, (_RS, DH))
    emb = emb_ref[...]
    v0_ref[...] = jnp.stack([a * emb[:, :DH], a * emb[:, DH:]], axis=0)


def _setup_call(ds_col, dd_col, emb):
    grid = (pl.cdiv(NP, _RS),)
    return pl.pallas_call(
        _setup_body,
        grid=grid,
        in_specs=[
            pl.BlockSpec((_RS, 1), lambda r: (r, 0)),
            pl.BlockSpec((_RS, 1), lambda r: (r, 0)),
            pl.BlockSpec((_RS, D), lambda r: (r, 0)),
        ],
        out_specs=[
            pl.BlockSpec((_RS, DH), lambda r: (r, 0)),
            pl.BlockSpec((_RS, DH), lambda r: (r, 0)),
            pl.BlockSpec((NC, _RS, DH), lambda r: (0, r, 0)),
        ],
        out_shape=[
            jax.ShapeDtypeStruct((NP, DH), jnp.float32),
            jax.ShapeDtypeStruct((NP, DH), jnp.float32),
            jax.ShapeDtypeStruct((NC, NP, DH), jnp.float32),
        ],
    )(ds_col, dd_col, emb)


_FR = 1564
_FC = 1024
_RB = 68


def _mid_body(s_ref, acc_ref, b_ref, c_ref, accn_ref, vn_ref):
    sv = s_ref[...]
    accn_ref[...] = acc_ref[...] + b_ref[...][None] * sv
    vn_ref[...] = c_ref[...][None] * sv


def _mid_call(s_f, acc_f, bexp_f, cexp_f):
    grid = (NC, _FR // _RB)
    blk3 = pl.BlockSpec((1, _RB, _FC), lambda c, r: (c, r, 0))
    blk2 = pl.BlockSpec((_RB, _FC), lambda c, r: (r, 0))
    return pl.pallas_call(
        _mid_body,
        grid=grid,
        in_specs=[blk3, blk3, blk2, blk2],
        out_specs=[blk3, blk3],
        out_shape=[
            jax.ShapeDtypeStruct((NC, _FR, _FC), jnp.float32),
            jax.ShapeDtypeStruct((NC, _FR, _FC), jnp.float32),
        ],
    )(s_f, acc_f, bexp_f, cexp_f)


def _final_body(s_ref, acc_ref, b_ref, out_ref):
    out_ref[...] = (acc_ref[...][0] + b_ref[...] * s_ref[...][0]) * 0.25


def _final_call(s3, acc3, bexp):
    grid = (NC, pl.cdiv(NP, _RS))
    blk3 = pl.BlockSpec((1, _RS, DH), lambda c, r: (c, r, 0))
    blk2 = pl.BlockSpec((_RS, DH), lambda c, r: (r, 0))
    return pl.pallas_call(
        _final_body,
        grid=grid,
        in_specs=[blk3, blk3, blk2],
        out_specs=pl.BlockSpec((_RS, DH), lambda c, r: (r, c)),
        out_shape=jax.ShapeDtypeStruct((NP, D), jnp.float32),
    )(s3, acc3, bexp)


def kernel(edge_index, user_table, item_table):
    src = edge_index[0].reshape(NT, E_TILE)
    dst = edge_index[1].reshape(NT, E_TILE)
    pad = jnp.full((NT, EP - E_TILE), N, jnp.int32)
    idx2 = jnp.stack(
        [jnp.concatenate([src, pad], axis=1), jnp.concatenate([dst, pad], axis=1)]
    )

    emb = jnp.concatenate([user_table, item_table], axis=0)
    emb = jnp.concatenate([emb, jnp.zeros((NP - N, D), jnp.float32)], axis=0)

    z1 = jnp.zeros((NP,), jnp.float32)
    z2 = jnp.zeros((NP, DH), jnp.float32)

    deg2 = _deg_call(idx2, z1)
    bexp, cexp, v0 = _setup_call(
        deg2[0].reshape(NP, 1), deg2[1].reshape(NP, 1), emb
    )

    bexp_f = bexp.reshape(_FR, _FC)
    cexp_f = cexp.reshape(_FR, _FC)

    acc = jnp.stack([emb[:, :DH], emb[:, DH:]], axis=0)
    v = v0
    for _ in range(LAYERS - 1):
        sseg = _layer_call(idx2, v, z2)
        acc_f, v_f = _mid_call(
            sseg.reshape(NC, _FR, _FC), acc.reshape(NC, _FR, _FC), bexp_f, cexp_f
        )
        acc = acc_f.reshape(NC, NP, DH)
        v = v_f.reshape(NC, NP, DH)

    sseg = _layer_call(idx2, v, z2)
    out = _final_call(sseg, acc, bexp)
    return out[:N]


# speedup vs baseline: 7.5640x; 7.5640x over previous
import jax
import jax.numpy as jnp
from jax import lax
from jax.experimental import pallas as pl
from jax.experimental.pallas import tpu as pltpu
from jax.experimental.pallas import tpu_sc as plsc

N_USERS = 25000
M_ITEMS = 25000
N = N_USERS + M_ITEMS
D = 64
DH = D // 2
LAYERS = 3
E = 800000

NC = 2
NT = 16
B = 128
E_TILE = E // NT
NBLK = -(-E_TILE // B)
EP = NBLK * B
NP = 50048
ROWS_TILE = NP // NT


def _sc_mesh():
    return plsc.VectorSubcoreMesh(
        core_axis_name="c", subcore_axis_name="s", num_cores=NC, num_subcores=NT
    )


def _deg_body(idx2, z1, deg_out, degacc, idx_v, ones_v):
    c = lax.axis_index("c")
    s = lax.axis_index("s")
    for i in range(B // 16):
        ones_v[pl.ds(i * 16, 16)] = jnp.full((16,), 1.0, jnp.float32)
    @pl.when(s == 0)
    def _init():
        pltpu.sync_copy(z1, degacc)

    plsc.subcore_barrier()

    def blk(i, carry):
        pltpu.sync_copy(idx2.at[c, s, pl.ds(i * B, B)], idx_v)
        pltpu.sync_copy(ones_v, degacc.at[idx_v], add=True)
        return carry

    lax.fori_loop(0, NBLK, blk, 0)
    plsc.subcore_barrier()

    @pl.when(s == 0)
    def _writeback():
        pltpu.sync_copy(degacc, deg_out.at[c, 0])


def _deg_call(idx2, z1):
    return pl.kernel(
        _deg_body,
        out_type=jax.ShapeDtypeStruct((NC, 1, NP), jnp.float32),
        mesh=_sc_mesh(),
        compiler_params=pltpu.CompilerParams(use_tc_tiling_on_sc=False),
        scratch_types=[
            pltpu.VMEM_SHARED((NP,), jnp.float32),
            pltpu.VMEM((B,), jnp.int32),
            pltpu.VMEM((B,), jnp.float32),
        ],
    )(idx2, z1)


def _layer_body(idx2, v, z2, seg_out, acc, src_v, dst_v, rows, sem):
    c = lax.axis_index("c")
    s = lax.axis_index("s")
    r0 = s * ROWS_TILE
    pltpu.sync_copy(z2.at[pl.ds(r0, ROWS_TILE)], acc.at[pl.ds(r0, ROWS_TILE)])
    plsc.subcore_barrier()

    def blk(i, carry):
        pltpu.sync_copy(idx2.at[0, s, pl.ds(i * B, B)], src_v)
        pltpu.sync_copy(idx2.at[1, s, pl.ds(i * B, B)], dst_v)
        pltpu.async_copy(v.at[c].at[src_v], rows, sem).wait()
        pltpu.sync_copy(rows, acc.at[dst_v], add=True)
        return carry

    lax.fori_loop(0, NBLK, blk, 0)
    plsc.subcore_barrier()
    pltpu.sync_copy(acc.at[pl.ds(r0, ROWS_TILE)], seg_out.at[c, pl.ds(r0, ROWS_TILE)])


def _layer_call(idx2, v, z2):
    return pl.kernel(
        _layer_body,
        out_type=jax.ShapeDtypeStruct((NC, NP, DH), jnp.float32),
        mesh=_sc_mesh(),
        compiler_params=pltpu.CompilerParams(use_tc_tiling_on_sc=False),
        scratch_types=[
            pltpu.VMEM_SHARED((NP, DH), jnp.float32),
            pltpu.VMEM((B,), jnp.int32),
            pltpu.VMEM((B,), jnp.int32),
            pltpu.VMEM((B, DH), jnp.float32),
            pltpu.SemaphoreType.DMA,
        ],
    )(idx2, v, z2)


_RS = 512


def _setup_body(ds_ref, dd_ref, emb_ref, bexp_ref, cexp_ref, v0_ref):
    a = lax.rsqrt(jnp.maximum(ds_ref[...], 1.0))
    b = lax.rsqrt(jnp.maximum(dd_ref[...], 1.0))
    bexp_ref[...] = jnp.broadcast_to(b, (_RS, DH))
    cexp_ref[...] = jnp.broadcast_to(a * b, (_RS, DH))
    emb = emb_ref[...]
    v0_ref[...] = jnp.stack([a * emb[:, :DH], a * emb[:, DH:]], axis=0)


def _setup_call(ds_col, dd_col, emb):
    grid = (pl.cdiv(NP, _RS),)
    return pl.pallas_call(
        _setup_body,
        grid=grid,
        in_specs=[
            pl.BlockSpec((_RS, 1), lambda r: (r, 0)),
            pl.BlockSpec((_RS, 1), lambda r: (r, 0)),
            pl.BlockSpec((_RS, D), lambda r: (r, 0)),
        ],
        out_specs=[
            pl.BlockSpec((_RS, DH), lambda r: (r, 0)),
            pl.BlockSpec((_RS, DH), lambda r: (r, 0)),
            pl.BlockSpec((NC, _RS, DH), lambda r: (0, r, 0)),
        ],
        out_shape=[
            jax.ShapeDtypeStruct((NP, DH), jnp.float32),
            jax.ShapeDtypeStruct((NP, DH), jnp.float32),
            jax.ShapeDtypeStruct((NC, NP, DH), jnp.float32),
        ],
    )(ds_col, dd_col, emb)


_FR = 3128
_FC = 512
_RB = 136


def _mid_body(s_ref, acc_ref, b_ref, c_ref, accn_ref, vn_ref):
    sv = s_ref[...]
    accn_ref[...] = acc_ref[...] + b_ref[...][None] * sv
    vn_ref[...] = c_ref[...][None] * sv


def _mid_call(s_f, acc_f, bexp_f, cexp_f):
    grid = (NC, _FR // _RB)
    blk3 = pl.BlockSpec((1, _RB, _FC), lambda c, r: (c, r, 0))
    blk2 = pl.BlockSpec((_RB, _FC), lambda c, r: (r, 0))
    return pl.pallas_call(
        _mid_body,
        grid=grid,
        in_specs=[blk3, blk3, blk2, blk2],
        out_specs=[blk3, blk3],
        out_shape=[
            jax.ShapeDtypeStruct((NC, _FR, _FC), jnp.float32),
            jax.ShapeDtypeStruct((NC, _FR, _FC), jnp.float32),
        ],
    )(s_f, acc_f, bexp_f, cexp_f)


def _final_body(s_ref, acc_ref, b_ref, out_ref):
    s = s_ref[...]
    acc = acc_ref[...]
    b = b_ref[...]
    res = (acc + b[None] * s) * 0.25
    out_ref[...] = jnp.concatenate([res[0], res[1]], axis=1)


def _final_call(s3, acc3, bexp):
    grid = (pl.cdiv(NP, _RS),)
    blk3 = pl.BlockSpec((NC, _RS, DH), lambda r: (0, r, 0))
    blk2 = pl.BlockSpec((_RS, DH), lambda r: (r, 0))
    return pl.pallas_call(
        _final_body,
        grid=grid,
        in_specs=[blk3, blk3, blk2],
        out_specs=pl.BlockSpec((_RS, D), lambda r: (r, 0)),
        out_shape=jax.ShapeDtypeStruct((NP, D), jnp.float32),
    )(s3, acc3, bexp)


def kernel(edge_index, user_table, item_table):
    src = edge_index[0].reshape(NT, E_TILE)
    dst = edge_index[1].reshape(NT, E_TILE)
    pad = jnp.full((NT, EP - E_TILE), N, jnp.int32)
    idx2 = jnp.stack(
        [jnp.concatenate([src, pad], axis=1), jnp.concatenate([dst, pad], axis=1)]
    )

    emb = jnp.concatenate([user_table, item_table], axis=0)
    emb = jnp.concatenate([emb, jnp.zeros((NP - N, D), jnp.float32)], axis=0)

    z1 = jnp.zeros((NP,), jnp.float32)
    z2 = jnp.zeros((NP, DH), jnp.float32)

    deg2 = _deg_call(idx2, z1)
    bexp, cexp, v0 = _setup_call(
        deg2[0].reshape(NP, 1), deg2[1].reshape(NP, 1), emb
    )

    bexp_f = bexp.reshape(_FR, _FC)
    cexp_f = cexp.reshape(_FR, _FC)

    acc = jnp.stack([emb[:, :DH], emb[:, DH:]], axis=0)
    v = v0
    for _ in range(LAYERS - 1):
        sseg = _layer_call(idx2, v, z2)
        acc_f, v_f = _mid_call(
            sseg.reshape(NC, _FR, _FC), acc.reshape(NC, _FR, _FC), bexp_f, cexp_f
        )
        acc = acc_f.reshape(NC, NP, DH)
        v = v_f.reshape(NC, NP, DH)

    sseg = _layer_call(idx2, v, z2)
    out = _final_call(sseg, acc, bexp)
    return out[:N]


# speedup vs baseline: 10.4007x; 1.3750x over previous
import jax
import jax.numpy as jnp
from jax import lax
from jax.experimental import pallas as pl
from jax.experimental.pallas import tpu as pltpu
from jax.experimental.pallas import tpu_sc as plsc

N_USERS = 25000
M_ITEMS = 25000
N = N_USERS + M_ITEMS
D = 64
LAYERS = 3
E = 800000

NC = 2
NT = 16
B = 128

NP = 50176
HALF = NP // 2
HALFP = HALF + 8
RT = HALF // NT

KS = 2
NCHS = 198
EPS = NCHS * KS * B
E_TILE = E // NT
PB = EPS // B

NQ = 6
NR = 3

_GRP = B // 16


def _sc_mesh():
    return plsc.VectorSubcoreMesh(
        core_axis_name="c", subcore_axis_name="s", num_cores=NC, num_subcores=NT
    )


def _part_body(idx2, z1, deg_out, plist, counts, degacc, srcbuf, dstbuf,
               isrc, idst, ones_v, cnt_v,
               semi0, semi1, semi2, semd0, semd1, semd2):
    c = lax.axis_index("c")
    s = lax.axis_index("s")
    tidx = c * NT + s
    semi = (semi0, semi1, semi2)
    semd = (semd0, semd1, semd2)
    lo = c * HALF
    lov = jnp.full((16,), lo, jnp.int32)
    hiv = lov + HALF
    zero16 = jnp.zeros((16,), jnp.int32)

    for i in range(B // 16):
        ones_v[pl.ds(i * 16, 16)] = jnp.full((16,), 1.0, jnp.float32)

    @pl.when(s == 0)
    def _init():
        pltpu.sync_copy(z1, degacc)

    def fill(i, carry):
        for k in range(_GRP):
            srcbuf[i, 0, pl.ds(k * 16, 16)] = jnp.full((16,), N, jnp.int32)
            dstbuf[i, 0, pl.ds(k * 16, 16)] = jnp.full((16,), HALF, jnp.int32)
        return carry

    lax.fori_loop(0, PB, fill, 0)
    plsc.subcore_barrier()

    base = s * NCHS
    pltpu.async_copy(idx2.at[0, base], isrc.at[0], semi[0])
    pltpu.async_copy(idx2.at[1, base], idst.at[0], semi[0])

    def outer(t, w):
        for j in range(3):
            g = t * 3 + j
            jn = (j + 1) % 3

            @pl.when(g >= 2)
            def _drain_deg():
                for _ in range(KS):
                    pltpu.make_async_copy(
                        ones_v, degacc.at[isrc.at[jn, 0]], semd[jn]
                    ).wait()

            @pl.when(g + 1 < NCHS)
            def _prefetch():
                pltpu.async_copy(idx2.at[0, base + g + 1], isrc.at[jn], semi[jn])
                pltpu.async_copy(idx2.at[1, base + g + 1], idst.at[jn], semi[jn])

            pltpu.make_async_copy(idx2.at[0, base + g], isrc.at[j], semi[j]).wait()
            pltpu.make_async_copy(idx2.at[1, base + g], idst.at[j], semi[j]).wait()

            for b in range(KS):
                @pl.when(c == 0)
                def _deg_src():
                    pltpu.async_copy(
                        ones_v, degacc.at[isrc.at[j, b]], semd[j], add=True
                    )

                @pl.when(c == 1)
                def _deg_dst():
                    pltpu.async_copy(
                        ones_v, degacc.at[idst.at[j, b]], semd[j], add=True
                    )

                for k in range(_GRP):
                    srcv = isrc[j, b, pl.ds(k * 16, 16)]
                    dstv = idst[j, b, pl.ds(k * 16, 16)]
                    m = jnp.logical_and(dstv >= lov, dstv < hiv)
                    mi = jnp.where(m, 1, 0).astype(jnp.int32)
                    wv = jnp.full((16,), w, jnp.int32)
                    pos = wv + plsc.cumsum(mi) - jnp.full((16,), 1, jnp.int32)
                    ph = jnp.right_shift(pos, jnp.full((16,), 7, jnp.int32))
                    pll = jnp.bitwise_and(pos, jnp.full((16,), 127, jnp.int32))
                    plsc.store_scatter(srcbuf, [ph, zero16, pll], srcv, mask=m)
                    plsc.store_scatter(dstbuf, [ph, zero16, pll], dstv - lov, mask=m)
                    w = w + jnp.sum(mi)
        return w

    w = lax.fori_loop(0, NCHS // 3, outer, jnp.int32(0))

    for slot in (1, 2):
        for _ in range(KS):
            pltpu.make_async_copy(
                ones_v, degacc.at[isrc.at[slot, 0]], semd[slot]
            ).wait()

    cnt_v[pl.ds(0, 16)] = jnp.full((16,), w, jnp.int32)
    pltpu.sync_copy(cnt_v, counts.at[tidx, 0])
    pltpu.sync_copy(srcbuf, plist.at[0, tidx])
    pltpu.sync_copy(dstbuf, plist.at[1, tidx])
    plsc.subcore_barrier()

    @pl.when(s == 0)
    def _writeback():
        pltpu.sync_copy(degacc, deg_out.at[c, 0])


def _part_call(idx2, z1):
    return pl.kernel(
        _part_body,
        out_type=(
            jax.ShapeDtypeStruct((NC, 1, NP), jnp.float32),
            jax.ShapeDtypeStruct((2, NC * NT, PB, 1, B), jnp.int32),
            jax.ShapeDtypeStruct((NC * NT, 1, 16), jnp.int32),
        ),
        mesh=_sc_mesh(),
        compiler_params=pltpu.CompilerParams(use_tc_tiling_on_sc=False, needs_layout_passes=False),
        scratch_types=[
            pltpu.VMEM_SHARED((NP,), jnp.float32),
            pltpu.VMEM((PB, 1, B), jnp.int32),
            pltpu.VMEM((PB, 1, B), jnp.int32),
            pltpu.VMEM((3, KS, B), jnp.int32),
            pltpu.VMEM((3, KS, B), jnp.int32),
            pltpu.VMEM((B,), jnp.float32),
            pltpu.VMEM((16,), jnp.int32),
        ] + [pltpu.SemaphoreType.DMA] * 6,
    )(idx2, z1)


def _layer_body(plist, counts, v, z2, seg_out, acc, src_c, dst_c, rows, cnt_v,
                semi0, semi1, semi2, semi3, semi4, semi5,
                semg0, semg1, semg2, sems0, sems1, sems2):
    c = lax.axis_index("c")
    s = lax.axis_index("s")
    tidx = c * NT + s
    semi = (semi0, semi1, semi2, semi3, semi4, semi5)
    semg = (semg0, semg1, semg2)
    sems_ = (sems0, sems1, sems2)

    r0 = s * RT
    pltpu.sync_copy(z2.at[pl.ds(r0, RT)], acc.at[pl.ds(r0, RT)])
    pltpu.sync_copy(counts.at[tidx, 0], cnt_v)
    plsc.subcore_barrier()

    w = jnp.max(cnt_v[pl.ds(0, 16)])
    nb6 = jnp.right_shift((jnp.right_shift(w + 127, 7) + 5) * 2731, 14)
    ntrip = jnp.maximum(nb6, 1)

    for q in range(2):
        pltpu.async_copy(plist.at[0, tidx, q, 0], src_c.at[q, 0], semi[q])
        pltpu.async_copy(plist.at[1, tidx, q, 0], dst_c.at[q, 0], semi[q])

    def outer(t, carry):
        for j in range(NQ):
            i = t * NQ + j
            r = j % NR
            rp = (j - 1) % NR
            qn = (j + 2) % NQ
            qp = (j - 1) % NQ

            @pl.when(i >= 3)
            def _drain_scatter():
                pltpu.make_async_copy(
                    rows.at[r], acc.at[dst_c.at[j, 0]], sems_[r]
                ).wait()

            @pl.when(i + 2 < ntrip * NQ)
            def _prefetch():
                pltpu.async_copy(
                    plist.at[0, tidx, i + 2, 0], src_c.at[qn, 0], semi[qn]
                )
                pltpu.async_copy(
                    plist.at[1, tidx, i + 2, 0], dst_c.at[qn, 0], semi[qn]
                )

            pltpu.make_async_copy(
                plist.at[0, tidx, i, 0], src_c.at[j, 0], semi[j]
            ).wait()
            pltpu.make_async_copy(
                plist.at[1, tidx, i, 0], dst_c.at[j, 0], semi[j]
            ).wait()

            pltpu.async_copy(v.at[src_c.at[j, 0]], rows.at[r], semg[r])

            @pl.when(i >= 1)
            def _advance_prev():
                pltpu.make_async_copy(
                    v.at[src_c.at[qp, 0]], rows.at[rp], semg[rp]
                ).wait()
                pltpu.async_copy(
                    rows.at[rp], acc.at[dst_c.at[qp, 0]], sems_[rp], add=True
                )
        return carry

    lax.fori_loop(0, ntrip, outer, 0)

    pltpu.make_async_copy(
        v.at[src_c.at[NQ - 1, 0]], rows.at[(NQ - 1) % NR], semg[(NQ - 1) % NR]
    ).wait()
    pltpu.async_copy(
        rows.at[(NQ - 1) % NR], acc.at[dst_c.at[NQ - 1, 0]],
        sems_[(NQ - 1) % NR], add=True,
    )
    for slot, p in ((3, 0), (4, 1), (5, 2)):
        pltpu.make_async_copy(rows.at[p], acc.at[dst_c.at[slot, 0]], sems_[p]).wait()

    plsc.subcore_barrier()
    pltpu.sync_copy(
        acc.at[pl.ds(r0, RT)], seg_out.at[pl.ds(c * HALF + r0, RT)]
    )


def _layer_call(plist, counts, v, z2):
    return pl.kernel(
        _layer_body,
        out_type=jax.ShapeDtypeStruct((NP, D), jnp.float32),
        mesh=_sc_mesh(),
        compiler_params=pltpu.CompilerParams(use_tc_tiling_on_sc=False, needs_layout_passes=False),
        scratch_types=[
            pltpu.VMEM_SHARED((HALFP, D), jnp.float32),
            pltpu.VMEM((NQ, 1, B), jnp.int32),
            pltpu.VMEM((NQ, 1, B), jnp.int32),
            pltpu.VMEM((NR, B, D), jnp.float32),
            pltpu.VMEM((16,), jnp.int32),
        ] + [pltpu.SemaphoreType.DMA] * 12,
    )(plist, counts, v, z2)


_RS = 512


def _setup_body(ds_ref, dd_ref, emb_ref, bexp_ref, cexp_ref, v0_ref):
    a = lax.rsqrt(jnp.maximum(ds_ref[...], 1.0))
    b = lax.rsqrt(jnp.maximum(dd_ref[...], 1.0))
    bexp_ref[...] = jnp.broadcast_to(b, (_RS, D))
    cexp_ref[...] = jnp.broadcast_to(a * b, (_RS, D))
    v0_ref[...] = a * emb_ref[...]


def _setup_call(ds_col, dd_col, emb):
    grid = (NP // _RS,)
    return pl.pallas_call(
        _setup_body,
        grid=grid,
        in_specs=[
            pl.BlockSpec((_RS, 1), lambda r: (r, 0)),
            pl.BlockSpec((_RS, 1), lambda r: (r, 0)),
            pl.BlockSpec((_RS, D), lambda r: (r, 0)),
        ],
        out_specs=[
            pl.BlockSpec((_RS, D), lambda r: (r, 0)),
            pl.BlockSpec((_RS, D), lambda r: (r, 0)),
            pl.BlockSpec((_RS, D), lambda r: (r, 0)),
        ],
        out_shape=[
            jax.ShapeDtypeStruct((NP, D), jnp.float32),
            jax.ShapeDtypeStruct((NP, D), jnp.float32),
            jax.ShapeDtypeStruct((NP, D), jnp.float32),
        ],
    )(ds_col, dd_col, emb)


_FR = 6272
_FC = 512
_RB = 392


def _mid_body(s_ref, acc_ref, b_ref, c_ref, accn_ref, vn_ref):
    sv = s_ref[...]
    accn_ref[...] = acc_ref[...] + b_ref[...] * sv
    vn_ref[...] = c_ref[...] * sv


def _mid_call(s_f, acc_f, bexp_f, cexp_f):
    grid = (_FR // _RB,)
    blk = pl.BlockSpec((_RB, _FC), lambda r: (r, 0))
    return pl.pallas_call(
        _mid_body,
        grid=grid,
        in_specs=[blk, blk, blk, blk],
        out_specs=[blk, blk],
        out_shape=[
            jax.ShapeDtypeStruct((_FR, _FC), jnp.float32),
            jax.ShapeDtypeStruct((_FR, _FC), jnp.float32),
        ],
    )(s_f, acc_f, bexp_f, cexp_f)


def _final_body(s_ref, acc_ref, b_ref, out_ref):
    out_ref[...] = (acc_ref[...] + b_ref[...] * s_ref[...]) * 0.25


def _final_call(s_f, acc_f, bexp_f):
    grid = (_FR // _RB,)
    blk = pl.BlockSpec((_RB, _FC), lambda r: (r, 0))
    return pl.pallas_call(
        _final_body,
        grid=grid,
        in_specs=[blk, blk, blk],
        out_specs=blk,
        out_shape=jax.ShapeDtypeStruct((_FR, _FC), jnp.float32),
    )(s_f, acc_f, bexp_f)


def kernel(edge_index, user_table, item_table):
    src = edge_index[0].reshape(NT, E_TILE)
    dst = edge_index[1].reshape(NT, E_TILE)
    pad = jnp.full((NT, EPS - E_TILE), N, jnp.int32)
    idx2 = jnp.stack(
        [jnp.concatenate([src, pad], axis=1), jnp.concatenate([dst, pad], axis=1)]
    ).reshape(2, NT * NCHS, KS, B)

    emb = jnp.concatenate([user_table, item_table], axis=0)
    emb = jnp.concatenate([emb, jnp.zeros((NP - N, D), jnp.float32)], axis=0)

    z1 = jnp.zeros((NP,), jnp.float32)
    z2 = jnp.zeros((HALF, D), jnp.float32)

    deg2, plist, counts = _part_call(idx2, z1)
    bexp, cexp, v = _setup_call(
        deg2[0].reshape(NP, 1), deg2[1].reshape(NP, 1), emb
    )

    bexp_f = bexp.reshape(_FR, _FC)
    cexp_f = cexp.reshape(_FR, _FC)

    acc_f = emb.reshape(_FR, _FC)
    for _ in range(LAYERS - 1):
        sseg = _layer_call(plist, counts, v, z2)
        acc_f, v_f = _mid_call(sseg.reshape(_FR, _FC), acc_f, bexp_f, cexp_f)
        v = v_f.reshape(NP, D)

    sseg = _layer_call(plist, counts, v, z2)
    out = _final_call(sseg.reshape(_FR, _FC), acc_f, bexp_f)
    return out.reshape(NP, D)[:N]


# speedup vs baseline: 11.4296x; 1.0989x over previous
import jax
import jax.numpy as jnp
from jax import lax
from jax.experimental import pallas as pl
from jax.experimental.pallas import tpu as pltpu
from jax.experimental.pallas import tpu_sc as plsc

N_USERS = 25000
M_ITEMS = 25000
N = N_USERS + M_ITEMS
D = 64
LAYERS = 3
E = 800000

NC = 2
NT = 16
B = 128

NP = 50176
HALF = NP // 2
HALFP = HALF + 8
RT = HALF // NT

KS = 2
NCHS = 198
EPS = NCHS * KS * B
E_TILE = E // NT
PB = EPS // B

NQ = 6
NR = 3

_GRP = B // 16


def _sc_mesh():
    return plsc.VectorSubcoreMesh(
        core_axis_name="c", subcore_axis_name="s", num_cores=NC, num_subcores=NT
    )


def _part_body(idx2, z1, deg_out, plist, counts, degacc, srcbuf, dstbuf,
               isrc, idst, ones_v, cnt_v,
               semi0, semi1, semi2, semd0, semd1, semd2):
    c = lax.axis_index("c")
    s = lax.axis_index("s")
    tidx = c * NT + s
    semi = (semi0, semi1, semi2)
    semd = (semd0, semd1, semd2)
    lo = c * HALF
    lov = jnp.full((16,), lo, jnp.int32)
    hiv = lov + HALF
    zero16 = jnp.zeros((16,), jnp.int32)

    for i in range(B // 16):
        ones_v[pl.ds(i * 16, 16)] = jnp.full((16,), 1.0, jnp.float32)

    @pl.when(s == 0)
    def _init():
        pltpu.sync_copy(z1, degacc)

    def fill(i, carry):
        for k in range(_GRP):
            srcbuf[i, 0, pl.ds(k * 16, 16)] = jnp.full((16,), N, jnp.int32)
            dstbuf[i, 0, pl.ds(k * 16, 16)] = jnp.full((16,), HALF, jnp.int32)
        return carry

    lax.fori_loop(0, PB, fill, 0)
    plsc.subcore_barrier()

    base = s * NCHS
    pltpu.async_copy(idx2.at[0, base], isrc.at[0], semi[0])
    pltpu.async_copy(idx2.at[1, base], idst.at[0], semi[0])

    def outer(t, w):
        for j in range(3):
            g = t * 3 + j
            jn = (j + 1) % 3

            @pl.when(g >= 2)
            def _drain_deg():
                for _ in range(KS):
                    pltpu.make_async_copy(
                        ones_v, degacc.at[isrc.at[jn, 0]], semd[jn]
                    ).wait()

            @pl.when(g + 1 < NCHS)
            def _prefetch():
                pltpu.async_copy(idx2.at[0, base + g + 1], isrc.at[jn], semi[jn])
                pltpu.async_copy(idx2.at[1, base + g + 1], idst.at[jn], semi[jn])

            pltpu.make_async_copy(idx2.at[0, base + g], isrc.at[j], semi[j]).wait()
            pltpu.make_async_copy(idx2.at[1, base + g], idst.at[j], semi[j]).wait()

            for b in range(KS):
                @pl.when(c == 0)
                def _deg_src():
                    pltpu.async_copy(
                        ones_v, degacc.at[isrc.at[j, b]], semd[j], add=True
                    )

                @pl.when(c == 1)
                def _deg_dst():
                    pltpu.async_copy(
                        ones_v, degacc.at[idst.at[j, b]], semd[j], add=True
                    )

                for k in range(_GRP):
                    srcv = isrc[j, b, pl.ds(k * 16, 16)]
                    dstv = idst[j, b, pl.ds(k * 16, 16)]
                    m = jnp.logical_and(dstv >= lov, dstv < hiv)
                    mi = jnp.where(m, 1, 0).astype(jnp.int32)
                    wv = jnp.full((16,), w, jnp.int32)
                    pos = wv + plsc.cumsum(mi) - jnp.full((16,), 1, jnp.int32)
                    ph = jnp.right_shift(pos, jnp.full((16,), 7, jnp.int32))
                    pll = jnp.bitwise_and(pos, jnp.full((16,), 127, jnp.int32))
                    plsc.store_scatter(srcbuf, [ph, zero16, pll], srcv, mask=m)
                    plsc.store_scatter(dstbuf, [ph, zero16, pll], dstv - lov, mask=m)
                    w = w + jnp.sum(mi)
        return w

    w = lax.fori_loop(0, NCHS // 3, outer, jnp.int32(0))

    for slot in (1, 2):
        for _ in range(KS):
            pltpu.make_async_copy(
                ones_v, degacc.at[isrc.at[slot, 0]], semd[slot]
            ).wait()

    cnt_v[pl.ds(0, 16)] = jnp.full((16,), w, jnp.int32)
    pltpu.sync_copy(cnt_v, counts.at[tidx, 0])
    pltpu.sync_copy(srcbuf, plist.at[0, tidx])
    pltpu.sync_copy(dstbuf, plist.at[1, tidx])
    plsc.subcore_barrier()

    @pl.when(s == 0)
    def _writeback():
        pltpu.sync_copy(degacc, deg_out.at[c, 0])


def _part_call(idx2, z1):
    return pl.kernel(
        _part_body,
        out_type=(
            jax.ShapeDtypeStruct((NC, 1, NP), jnp.float32),
            jax.ShapeDtypeStruct((2, NC * NT, PB, 1, B), jnp.int32),
            jax.ShapeDtypeStruct((NC * NT, 1, 16), jnp.int32),
        ),
        mesh=_sc_mesh(),
        compiler_params=pltpu.CompilerParams(use_tc_tiling_on_sc=False, needs_layout_passes=False),
        scratch_types=[
            pltpu.VMEM_SHARED((NP,), jnp.float32),
            pltpu.VMEM((PB, 1, B), jnp.int32),
            pltpu.VMEM((PB, 1, B), jnp.int32),
            pltpu.VMEM((3, KS, B), jnp.int32),
            pltpu.VMEM((3, KS, B), jnp.int32),
            pltpu.VMEM((B,), jnp.float32),
            pltpu.VMEM((16,), jnp.int32),
        ] + [pltpu.SemaphoreType.DMA] * 6,
    )(idx2, z1)


def _layer_body(plist, counts, v, z2, seg_out, acc, src_c, dst_c, rows, cnt_v,
                semi0, semi1, semi2, semi3, semi4, semi5,
                semg0, semg1, semg2, sems0, sems1, sems2):
    c = lax.axis_index("c")
    s = lax.axis_index("s")
    tidx = c * NT + s
    semi = (semi0, semi1, semi2, semi3, semi4, semi5)
    semg = (semg0, semg1, semg2)
    sems_ = (sems0, sems1, sems2)

    r0 = s * RT
    pltpu.sync_copy(z2.at[pl.ds(r0, RT)], acc.at[pl.ds(r0, RT)])
    pltpu.sync_copy(counts.at[tidx, 0], cnt_v)
    plsc.subcore_barrier()
    vc = v.at[c]

    w = jnp.max(cnt_v[pl.ds(0, 16)])
    nb6 = jnp.right_shift((jnp.right_shift(w + 127, 7) + 5) * 2731, 14)
    ntrip = jnp.maximum(nb6, 1)

    for q in range(2):
        pltpu.async_copy(plist.at[0, tidx, q, 0], src_c.at[q, 0], semi[q])
        pltpu.async_copy(plist.at[1, tidx, q, 0], dst_c.at[q, 0], semi[q])

    def outer(t, carry):
        for j in range(NQ):
            i = t * NQ + j
            r = j % NR
            rp = (j - 1) % NR
            qn = (j + 2) % NQ
            qp = (j - 1) % NQ

            @pl.when(i >= 3)
            def _drain_scatter():
                pltpu.make_async_copy(
                    rows.at[r], acc.at[dst_c.at[j, 0]], sems_[r]
                ).wait()

            @pl.when(i + 2 < ntrip * NQ)
            def _prefetch():
                pltpu.async_copy(
                    plist.at[0, tidx, i + 2, 0], src_c.at[qn, 0], semi[qn]
                )
                pltpu.async_copy(
                    plist.at[1, tidx, i + 2, 0], dst_c.at[qn, 0], semi[qn]
                )

            pltpu.make_async_copy(
                plist.at[0, tidx, i, 0], src_c.at[j, 0], semi[j]
            ).wait()
            pltpu.make_async_copy(
                plist.at[1, tidx, i, 0], dst_c.at[j, 0], semi[j]
            ).wait()

            pltpu.async_copy(vc.at[src_c.at[j, 0]], rows.at[r], semg[r])

            @pl.when(i >= 1)
            def _advance_prev():
                pltpu.make_async_copy(
                    vc.at[src_c.at[qp, 0]], rows.at[rp], semg[rp]
                ).wait()
                pltpu.async_copy(
                    rows.at[rp], acc.at[dst_c.at[qp, 0]], sems_[rp], add=True
                )
        return carry

    lax.fori_loop(0, ntrip, outer, 0)

    pltpu.make_async_copy(
        vc.at[src_c.at[NQ - 1, 0]], rows.at[(NQ - 1) % NR], semg[(NQ - 1) % NR]
    ).wait()
    pltpu.async_copy(
        rows.at[(NQ - 1) % NR], acc.at[dst_c.at[NQ - 1, 0]],
        sems_[(NQ - 1) % NR], add=True,
    )
    for slot, p in ((3, 0), (4, 1), (5, 2)):
        pltpu.make_async_copy(rows.at[p], acc.at[dst_c.at[slot, 0]], sems_[p]).wait()

    plsc.subcore_barrier()
    pltpu.sync_copy(
        acc.at[pl.ds(r0, RT)], seg_out.at[pl.ds(c * HALF + r0, RT)]
    )


def _layer_call(plist, counts, v, z2):
    return pl.kernel(
        _layer_body,
        out_type=jax.ShapeDtypeStruct((NP, D), jnp.float32),
        mesh=_sc_mesh(),
        compiler_params=pltpu.CompilerParams(use_tc_tiling_on_sc=False, needs_layout_passes=False),
        scratch_types=[
            pltpu.VMEM_SHARED((HALFP, D), jnp.float32),
            pltpu.VMEM((NQ, 1, B), jnp.int32),
            pltpu.VMEM((NQ, 1, B), jnp.int32),
            pltpu.VMEM((NR, B, D), jnp.float32),
            pltpu.VMEM((16,), jnp.int32),
        ] + [pltpu.SemaphoreType.DMA] * 12,
    )(plist, counts, v, z2)


_RS = 512


def _setup_body(ds_ref, dd_ref, emb_ref, bexp_ref, cexp_ref, v0_ref):
    a = lax.rsqrt(jnp.maximum(ds_ref[...], 1.0))
    b = lax.rsqrt(jnp.maximum(dd_ref[...], 1.0))
    bexp_ref[...] = jnp.broadcast_to(b, (_RS, D))
    cexp_ref[...] = jnp.broadcast_to(a * b, (_RS, D))
    av = a * emb_ref[...]
    v0_ref[...] = jnp.stack([av, av], axis=0)


def _setup_call(ds_col, dd_col, emb):
    grid = (NP // _RS,)
    return pl.pallas_call(
        _setup_body,
        grid=grid,
        in_specs=[
            pl.BlockSpec((_RS, 1), lambda r: (r, 0)),
            pl.BlockSpec((_RS, 1), lambda r: (r, 0)),
            pl.BlockSpec((_RS, D), lambda r: (r, 0)),
        ],
        out_specs=[
            pl.BlockSpec((_RS, D), lambda r: (r, 0)),
            pl.BlockSpec((_RS, D), lambda r: (r, 0)),
            pl.BlockSpec((NC, _RS, D), lambda r: (0, r, 0)),
        ],
        out_shape=[
            jax.ShapeDtypeStruct((NP, D), jnp.float32),
            jax.ShapeDtypeStruct((NP, D), jnp.float32),
            jax.ShapeDtypeStruct((NC, NP, D), jnp.float32),
        ],
    )(ds_col, dd_col, emb)


_FR = 6272
_FC = 512
_RB = 392


def _mid_body(s_ref, acc_ref, b_ref, c_ref, accn_ref, vn_ref):
    sv = s_ref[...]
    accn_ref[...] = acc_ref[...] + b_ref[...] * sv
    vnew = c_ref[...] * sv
    vn_ref[...] = jnp.stack([vnew, vnew], axis=0)


def _mid_call(s_f, acc_f, bexp_f, cexp_f):
    grid = (_FR // _RB,)
    blk = pl.BlockSpec((_RB, _FC), lambda r: (r, 0))
    return pl.pallas_call(
        _mid_body,
        grid=grid,
        in_specs=[blk, blk, blk, blk],
        out_specs=[blk, pl.BlockSpec((NC, _RB, _FC), lambda r: (0, r, 0))],
        out_shape=[
            jax.ShapeDtypeStruct((_FR, _FC), jnp.float32),
            jax.ShapeDtypeStruct((NC, _FR, _FC), jnp.float32),
        ],
    )(s_f, acc_f, bexp_f, cexp_f)


def _final_body(s_ref, acc_ref, b_ref, out_ref):
    out_ref[...] = (acc_ref[...] + b_ref[...] * s_ref[...]) * 0.25


def _final_call(s_f, acc_f, bexp_f):
    grid = (_FR // _RB,)
    blk = pl.BlockSpec((_RB, _FC), lambda r: (r, 0))
    return pl.pallas_call(
        _final_body,
        grid=grid,
        in_specs=[blk, blk, blk],
        out_specs=blk,
        out_shape=jax.ShapeDtypeStruct((_FR, _FC), jnp.float32),
    )(s_f, acc_f, bexp_f)


def kernel(edge_index, user_table, item_table):
    src = edge_index[0].reshape(NT, E_TILE)
    dst = edge_index[1].reshape(NT, E_TILE)
    pad = jnp.full((NT, EPS - E_TILE), N, jnp.int32)
    idx2 = jnp.stack(
        [jnp.concatenate([src, pad], axis=1), jnp.concatenate([dst, pad], axis=1)]
    ).reshape(2, NT * NCHS, KS, B)

    emb = jnp.concatenate([user_table, item_table], axis=0)
    emb = jnp.concatenate([emb, jnp.zeros((NP - N, D), jnp.float32)], axis=0)

    z1 = jnp.zeros((NP,), jnp.float32)
    z2 = jnp.zeros((HALF, D), jnp.float32)

    deg2, plist, counts = _part_call(idx2, z1)
    bexp, cexp, v = _setup_call(
        deg2[0].reshape(NP, 1), deg2[1].reshape(NP, 1), emb
    )

    bexp_f = bexp.reshape(_FR, _FC)
    cexp_f = cexp.reshape(_FR, _FC)

    acc_f = emb.reshape(_FR, _FC)
    for _ in range(LAYERS - 1):
        sseg = _layer_call(plist, counts, v, z2)
        acc_f, v_f = _mid_call(sseg.reshape(_FR, _FC), acc_f, bexp_f, cexp_f)
        v = v_f.reshape(NC, NP, D)

    sseg = _layer_call(plist, counts, v, z2)
    out = _final_call(sseg.reshape(_FR, _FC), acc_f, bexp_f)
    return out.reshape(NP, D)[:N]


# speedup vs baseline: 22.6710x; 1.9835x over previous
import jax
import jax.numpy as jnp
from jax import lax
from jax.experimental import pallas as pl
from jax.experimental.pallas import tpu as pltpu
from jax.experimental.pallas import tpu_sc as plsc

N_USERS = 25000
M_ITEMS = 25000
N = N_USERS + M_ITEMS
D = 64
LAYERS = 3
E = 800000

NC = 2
NT = 16
B = 128

NP = 50176
HALF = NP // 2
HALFP = HALF + 8
RT = HALF // NT

KS = 2
NCHS = 198
EPS = NCHS * KS * B
E_TILE = E // NT
PB = EPS // B

NQ = 6
NR = 3

_GRP = B // 16


def _sc_mesh():
    return plsc.VectorSubcoreMesh(
        core_axis_name="c", subcore_axis_name="s", num_cores=NC, num_subcores=NT
    )


def _part_body(idx2, z1, deg_out, plist, counts, degacc, srcbuf, dstbuf,
               isrc, idst, ones_v, cnt_v,
               semi0, semi1, semi2, semd0, semd1, semd2):
    c = lax.axis_index("c")
    s = lax.axis_index("s")
    tidx = c * NT + s
    semi = (semi0, semi1, semi2)
    semd = (semd0, semd1, semd2)
    lo = c * HALF
    lov = jnp.full((16,), lo, jnp.int32)
    hiv = lov + HALF
    zero16 = jnp.zeros((16,), jnp.int32)

    for i in range(B // 16):
        ones_v[pl.ds(i * 16, 16)] = jnp.full((16,), 1.0, jnp.float32)

    @pl.when(s == 0)
    def _init():
        pltpu.sync_copy(z1, degacc)

    iota16 = lax.iota(jnp.int32, 16)

    def fill(i, carry):
        sv = jnp.full((16,), N, jnp.int32) + jnp.bitwise_and(
            iota16 + jnp.full((16,), i, jnp.int32), jnp.full((16,), 127, jnp.int32)
        )
        dv = jnp.full((16,), HALF, jnp.int32) + jnp.bitwise_and(
            iota16, jnp.full((16,), 7, jnp.int32)
        )
        for k in range(_GRP):
            srcbuf[i, 0, pl.ds(k * 16, 16)] = sv
            dstbuf[i, 0, pl.ds(k * 16, 16)] = dv
        return carry

    lax.fori_loop(0, PB, fill, 0)
    plsc.subcore_barrier()

    base = s * NCHS
    pltpu.async_copy(idx2.at[0, base], isrc.at[0], semi[0])
    pltpu.async_copy(idx2.at[1, base], idst.at[0], semi[0])

    def outer(t, w):
        for j in range(3):
            g = t * 3 + j
            jn = (j + 1) % 3

            @pl.when(g >= 2)
            def _drain_deg():
                for _ in range(KS):
                    pltpu.make_async_copy(
                        ones_v, degacc.at[isrc.at[jn, 0]], semd[jn]
                    ).wait()

            @pl.when(g + 1 < NCHS)
            def _prefetch():
                pltpu.async_copy(idx2.at[0, base + g + 1], isrc.at[jn], semi[jn])
                pltpu.async_copy(idx2.at[1, base + g + 1], idst.at[jn], semi[jn])

            pltpu.make_async_copy(idx2.at[0, base + g], isrc.at[j], semi[j]).wait()
            pltpu.make_async_copy(idx2.at[1, base + g], idst.at[j], semi[j]).wait()

            for b in range(KS):
                @pl.when(c == 0)
                def _deg_src():
                    pltpu.async_copy(
                        ones_v, degacc.at[isrc.at[j, b]], semd[j], add=True
                    )

                @pl.when(c == 1)
                def _deg_dst():
                    pltpu.async_copy(
                        ones_v, degacc.at[idst.at[j, b]], semd[j], add=True
                    )

                for k in range(_GRP):
                    srcv = isrc[j, b, pl.ds(k * 16, 16)]
                    dstv = idst[j, b, pl.ds(k * 16, 16)]
                    m = jnp.logical_and(dstv >= lov, dstv < hiv)
                    mi = jnp.where(m, 1, 0).astype(jnp.int32)
                    wv = jnp.full((16,), w, jnp.int32)
                    pos = wv + plsc.cumsum(mi) - jnp.full((16,), 1, jnp.int32)
                    ph = jnp.right_shift(pos, jnp.full((16,), 7, jnp.int32))
                    pll = jnp.bitwise_and(pos, jnp.full((16,), 127, jnp.int32))
                    plsc.store_scatter(srcbuf, [ph, zero16, pll], srcv, mask=m)
                    plsc.store_scatter(dstbuf, [ph, zero16, pll], dstv - lov, mask=m)
                    w = w + jnp.sum(mi)
        return w

    w = lax.fori_loop(0, NCHS // 3, outer, jnp.int32(0))

    for slot in (1, 2):
        for _ in range(KS):
            pltpu.make_async_copy(
                ones_v, degacc.at[isrc.at[slot, 0]], semd[slot]
            ).wait()

    cnt_v[pl.ds(0, 16)] = jnp.full((16,), w, jnp.int32)
    pltpu.sync_copy(cnt_v, counts.at[tidx, 0])
    pltpu.sync_copy(srcbuf, plist.at[0, tidx])
    pltpu.sync_copy(dstbuf, plist.at[1, tidx])
    plsc.subcore_barrier()

    @pl.when(s == 0)
    def _writeback():
        pltpu.sync_copy(degacc, deg_out.at[c, 0])


def _part_call(idx2, z1):
    return pl.kernel(
        _part_body,
        out_type=(
            jax.ShapeDtypeStruct((NC, 1, NP), jnp.float32),
            jax.ShapeDtypeStruct((2, NC * NT, PB, 1, B), jnp.int32),
            jax.ShapeDtypeStruct((NC * NT, 1, 16), jnp.int32),
        ),
        mesh=_sc_mesh(),
        compiler_params=pltpu.CompilerParams(use_tc_tiling_on_sc=False, needs_layout_passes=False),
        scratch_types=[
            pltpu.VMEM_SHARED((NP,), jnp.float32),
            pltpu.VMEM((PB, 1, B), jnp.int32),
            pltpu.VMEM((PB, 1, B), jnp.int32),
            pltpu.VMEM((3, KS, B), jnp.int32),
            pltpu.VMEM((3, KS, B), jnp.int32),
            pltpu.VMEM((B,), jnp.float32),
            pltpu.VMEM((16,), jnp.int32),
        ] + [pltpu.SemaphoreType.DMA] * 6,
    )(idx2, z1)


def _layer_body(plist, counts, v, z2, seg_out, acc, src_c, dst_c, rows, cnt_v,
                semi0, semi1, semi2, semi3, semi4, semi5,
                semg0, semg1, semg2, sems0, sems1, sems2):
    c = lax.axis_index("c")
    s = lax.axis_index("s")
    tidx = c * NT + s
    semi = (semi0, semi1, semi2, semi3, semi4, semi5)
    semg = (semg0, semg1, semg2)
    sems_ = (sems0, sems1, sems2)

    r0 = s * RT
    pltpu.sync_copy(z2.at[pl.ds(r0, RT)], acc.at[pl.ds(r0, RT)])
    pltpu.sync_copy(counts.at[tidx, 0], cnt_v)
    plsc.subcore_barrier()
    vc = v.at[c]

    w = jnp.max(cnt_v[pl.ds(0, 16)])
    nb6 = jnp.right_shift((jnp.right_shift(w + 127, 7) + 5) * 2731, 14)
    ntrip = jnp.maximum(nb6, 1)

    for q in range(2):
        pltpu.async_copy(plist.at[0, tidx, q, 0], src_c.at[q, 0], semi[q])
        pltpu.async_copy(plist.at[1, tidx, q, 0], dst_c.at[q, 0], semi[q])

    def outer(t, carry):
        for j in range(NQ):
            i = t * NQ + j
            r = j % NR
            rp = (j - 1) % NR
            qn = (j + 2) % NQ
            qp = (j - 1) % NQ

            @pl.when(i >= 3)
            def _drain_scatter():
                pltpu.make_async_copy(
                    rows.at[r], acc.at[dst_c.at[j, 0]], sems_[r]
                ).wait()

            @pl.when(i + 2 < ntrip * NQ)
            def _prefetch():
                pltpu.async_copy(
                    plist.at[0, tidx, i + 2, 0], src_c.at[qn, 0], semi[qn]
                )
                pltpu.async_copy(
                    plist.at[1, tidx, i + 2, 0], dst_c.at[qn, 0], semi[qn]
                )

            pltpu.make_async_copy(
                plist.at[0, tidx, i, 0], src_c.at[j, 0], semi[j]
            ).wait()
            pltpu.make_async_copy(
                plist.at[1, tidx, i, 0], dst_c.at[j, 0], semi[j]
            ).wait()

            pltpu.async_copy(vc.at[src_c.at[j, 0]], rows.at[r], semg[r])

            @pl.when(i >= 1)
            def _advance_prev():
                pltpu.make_async_copy(
                    vc.at[src_c.at[qp, 0]], rows.at[rp], semg[rp]
                ).wait()
                pltpu.async_copy(
                    rows.at[rp], acc.at[dst_c.at[qp, 0]], sems_[rp], add=True
                )
        return carry

    lax.fori_loop(0, ntrip, outer, 0)

    pltpu.make_async_copy(
        vc.at[src_c.at[NQ - 1, 0]], rows.at[(NQ - 1) % NR], semg[(NQ - 1) % NR]
    ).wait()
    pltpu.async_copy(
        rows.at[(NQ - 1) % NR], acc.at[dst_c.at[NQ - 1, 0]],
        sems_[(NQ - 1) % NR], add=True,
    )
    for slot, p in ((3, 0), (4, 1), (5, 2)):
        pltpu.make_async_copy(rows.at[p], acc.at[dst_c.at[slot, 0]], sems_[p]).wait()

    plsc.subcore_barrier()
    pltpu.sync_copy(
        acc.at[pl.ds(r0, RT)], seg_out.at[pl.ds(c * HALF + r0, RT)]
    )


def _layer_call(plist, counts, v, z2):
    return pl.kernel(
        _layer_body,
        out_type=jax.ShapeDtypeStruct((NP, D), jnp.float32),
        mesh=_sc_mesh(),
        compiler_params=pltpu.CompilerParams(use_tc_tiling_on_sc=False, needs_layout_passes=False),
        scratch_types=[
            pltpu.VMEM_SHARED((HALFP, D), jnp.float32),
            pltpu.VMEM((NQ, 1, B), jnp.int32),
            pltpu.VMEM((NQ, 1, B), jnp.int32),
            pltpu.VMEM((NR, B, D), jnp.float32),
            pltpu.VMEM((16,), jnp.int32),
        ] + [pltpu.SemaphoreType.DMA] * 12,
    )(plist, counts, v, z2)


_RS = 512


def _setup_body(ds_ref, dd_ref, emb_ref, bexp_ref, cexp_ref, v0_ref):
    a = lax.rsqrt(jnp.maximum(ds_ref[...], 1.0))
    b = lax.rsqrt(jnp.maximum(dd_ref[...], 1.0))
    bexp_ref[...] = jnp.broadcast_to(b, (_RS, D))
    cexp_ref[...] = jnp.broadcast_to(a * b, (_RS, D))
    av = a * emb_ref[...]
    v0_ref[...] = jnp.stack([av, av], axis=0)


def _setup_call(ds_col, dd_col, emb):
    grid = (NP // _RS,)
    return pl.pallas_call(
        _setup_body,
        grid=grid,
        in_specs=[
            pl.BlockSpec((_RS, 1), lambda r: (r, 0)),
            pl.BlockSpec((_RS, 1), lambda r: (r, 0)),
            pl.BlockSpec((_RS, D), lambda r: (r, 0)),
        ],
        out_specs=[
            pl.BlockSpec((_RS, D), lambda r: (r, 0)),
            pl.BlockSpec((_RS, D), lambda r: (r, 0)),
            pl.BlockSpec((NC, _RS, D), lambda r: (0, r, 0)),
        ],
        out_shape=[
            jax.ShapeDtypeStruct((NP, D), jnp.float32),
            jax.ShapeDtypeStruct((NP, D), jnp.float32),
            jax.ShapeDtypeStruct((NC, NP, D), jnp.float32),
        ],
    )(ds_col, dd_col, emb)


_FR = 6272
_FC = 512
_RB = 392


def _mid_body(s_ref, acc_ref, b_ref, c_ref, accn_ref, vn_ref):
    sv = s_ref[...]
    accn_ref[...] = acc_ref[...] + b_ref[...] * sv
    vnew = c_ref[...] * sv
    vn_ref[...] = jnp.stack([vnew, vnew], axis=0)


def _mid_call(s_f, acc_f, bexp_f, cexp_f):
    grid = (_FR // _RB,)
    blk = pl.BlockSpec((_RB, _FC), lambda r: (r, 0))
    return pl.pallas_call(
        _mid_body,
        grid=grid,
        in_specs=[blk, blk, blk, blk],
        out_specs=[blk, pl.BlockSpec((NC, _RB, _FC), lambda r: (0, r, 0))],
        out_shape=[
            jax.ShapeDtypeStruct((_FR, _FC), jnp.float32),
            jax.ShapeDtypeStruct((NC, _FR, _FC), jnp.float32),
        ],
    )(s_f, acc_f, bexp_f, cexp_f)


def _final_body(s_ref, acc_ref, b_ref, out_ref):
    out_ref[...] = (acc_ref[...] + b_ref[...] * s_ref[...]) * 0.25


def _final_call(s_f, acc_f, bexp_f):
    grid = (_FR // _RB,)
    blk = pl.BlockSpec((_RB, _FC), lambda r: (r, 0))
    return pl.pallas_call(
        _final_body,
        grid=grid,
        in_specs=[blk, blk, blk],
        out_specs=blk,
        out_shape=jax.ShapeDtypeStruct((_FR, _FC), jnp.float32),
    )(s_f, acc_f, bexp_f)


def kernel(edge_index, user_table, item_table):
    src = edge_index[0].reshape(NT, E_TILE)
    dst = edge_index[1].reshape(NT, E_TILE)
    pad = N + (jnp.arange(EPS - E_TILE, dtype=jnp.int32) % 128)
    pad = jnp.broadcast_to(pad, (NT, EPS - E_TILE))
    idx2 = jnp.stack(
        [jnp.concatenate([src, pad], axis=1), jnp.concatenate([dst, pad], axis=1)]
    ).reshape(2, NT * NCHS, KS, B)

    emb = jnp.concatenate([user_table, item_table], axis=0)
    emb = jnp.concatenate([emb, jnp.zeros((NP - N, D), jnp.float32)], axis=0)

    z1 = jnp.zeros((NP,), jnp.float32)
    z2 = jnp.zeros((HALF, D), jnp.float32)

    deg2, plist, counts = _part_call(idx2, z1)
    bexp, cexp, v = _setup_call(
        deg2[0].reshape(NP, 1), deg2[1].reshape(NP, 1), emb
    )

    bexp_f = bexp.reshape(_FR, _FC)
    cexp_f = cexp.reshape(_FR, _FC)

    acc_f = emb.reshape(_FR, _FC)
    for _ in range(LAYERS - 1):
        sseg = _layer_call(plist, counts, v, z2)
        acc_f, v_f = _mid_call(sseg.reshape(_FR, _FC), acc_f, bexp_f, cexp_f)
        v = v_f.reshape(NC, NP, D)

    sseg = _layer_call(plist, counts, v, z2)
    out = _final_call(sseg.reshape(_FR, _FC), acc_f, bexp_f)
    return out.reshape(NP, D)[:N]


# speedup vs baseline: 23.1238x; 1.0200x over previous
import jax
import jax.numpy as jnp
from jax import lax
from jax.experimental import pallas as pl
from jax.experimental.pallas import tpu as pltpu
from jax.experimental.pallas import tpu_sc as plsc

N_USERS = 25000
M_ITEMS = 25000
N = N_USERS + M_ITEMS
D = 64
LAYERS = 3
E = 800000

NC = 2
NT = 16
B = 128

NP = 50176
HALF = NP // 2
HALFP = HALF + 8
RT = HALF // NT

KS = 2
NCHS = 198
EPS = NCHS * KS * B
E_TILE = E // NT
PB = EPS // B

NQ = 6
NR = 3

_GRP = B // 16


def _sc_mesh():
    return plsc.VectorSubcoreMesh(
        core_axis_name="c", subcore_axis_name="s", num_cores=NC, num_subcores=NT
    )


def _part_body(idx2, z1, deg_out, plist, counts, degacc, srcbuf, dstbuf,
               isrc, idst, ones_v, cnt_v,
               semi0, semi1, semi2, semd0, semd1, semd2):
    c = lax.axis_index("c")
    s = lax.axis_index("s")
    tidx = c * NT + s
    semi = (semi0, semi1, semi2)
    semd = (semd0, semd1, semd2)
    lo = c * HALF
    lov = jnp.full((16,), lo, jnp.int32)
    hiv = lov + HALF
    zero16 = jnp.zeros((16,), jnp.int32)

    for i in range(B // 16):
        ones_v[pl.ds(i * 16, 16)] = jnp.full((16,), 1.0, jnp.float32)

    @pl.when(s == 0)
    def _init():
        pltpu.sync_copy(z1, degacc)

    iota16 = lax.iota(jnp.int32, 16)

    def fill(i, carry):
        sv = jnp.full((16,), N, jnp.int32) + jnp.bitwise_and(
            iota16 + jnp.full((16,), i, jnp.int32), jnp.full((16,), 127, jnp.int32)
        )
        dv = jnp.full((16,), HALF, jnp.int32) + jnp.bitwise_and(
            iota16, jnp.full((16,), 7, jnp.int32)
        )
        for k in range(_GRP):
            srcbuf[i, 0, pl.ds(k * 16, 16)] = sv
            dstbuf[i, 0, pl.ds(k * 16, 16)] = dv
        return carry

    lax.fori_loop(0, PB, fill, 0)
    plsc.subcore_barrier()

    base = s * NCHS
    pltpu.async_copy(idx2.at[0, base], isrc.at[0], semi[0])
    pltpu.async_copy(idx2.at[1, base], idst.at[0], semi[0])

    def outer(t, w):
        for j in range(3):
            g = t * 3 + j
            jn = (j + 1) % 3

            @pl.when(g >= 2)
            def _drain_deg():
                for _ in range(KS):
                    pltpu.make_async_copy(
                        ones_v, degacc.at[isrc.at[jn, 0]], semd[jn]
                    ).wait()

            @pl.when(g + 1 < NCHS)
            def _prefetch():
                pltpu.async_copy(idx2.at[0, base + g + 1], isrc.at[jn], semi[jn])
                pltpu.async_copy(idx2.at[1, base + g + 1], idst.at[jn], semi[jn])

            pltpu.make_async_copy(idx2.at[0, base + g], isrc.at[j], semi[j]).wait()
            pltpu.make_async_copy(idx2.at[1, base + g], idst.at[j], semi[j]).wait()

            for b in range(KS):
                @pl.when(c == 0)
                def _deg_src():
                    pltpu.async_copy(
                        ones_v, degacc.at[isrc.at[j, b]], semd[j], add=True
                    )

                @pl.when(c == 1)
                def _deg_dst():
                    pltpu.async_copy(
                        ones_v, degacc.at[idst.at[j, b]], semd[j], add=True
                    )

                for k in range(_GRP):
                    srcv = isrc[j, b, pl.ds(k * 16, 16)]
                    dstv = idst[j, b, pl.ds(k * 16, 16)]
                    m = jnp.logical_and(dstv >= lov, dstv < hiv)
                    mi = jnp.where(m, 1, 0).astype(jnp.int32)
                    wv = jnp.full((16,), w, jnp.int32)
                    pos = wv + plsc.cumsum(mi) - jnp.full((16,), 1, jnp.int32)
                    ph = jnp.right_shift(pos, jnp.full((16,), 7, jnp.int32))
                    pll = jnp.bitwise_and(pos, jnp.full((16,), 127, jnp.int32))
                    plsc.store_scatter(srcbuf, [ph, zero16, pll], srcv, mask=m)
                    plsc.store_scatter(dstbuf, [ph, zero16, pll], dstv - lov, mask=m)
                    w = w + jnp.sum(mi)
        return w

    w = lax.fori_loop(0, NCHS // 3, outer, jnp.int32(0))

    for slot in (1, 2):
        for _ in range(KS):
            pltpu.make_async_copy(
                ones_v, degacc.at[isrc.at[slot, 0]], semd[slot]
            ).wait()

    cnt_v[pl.ds(0, 16)] = jnp.full((16,), w, jnp.int32)
    pltpu.sync_copy(cnt_v, counts.at[tidx, 0])
    pltpu.sync_copy(srcbuf, plist.at[0, tidx])
    pltpu.sync_copy(dstbuf, plist.at[1, tidx])
    plsc.subcore_barrier()

    @pl.when(s == 0)
    def _writeback():
        pltpu.sync_copy(degacc, deg_out.at[c, 0])


def _part_call(idx2, z1):
    return pl.kernel(
        _part_body,
        out_type=(
            jax.ShapeDtypeStruct((NC, 1, NP), jnp.float32),
            jax.ShapeDtypeStruct((2, NC * NT, PB, 1, B), jnp.int32),
            jax.ShapeDtypeStruct((NC * NT, 1, 16), jnp.int32),
        ),
        mesh=_sc_mesh(),
        compiler_params=pltpu.CompilerParams(use_tc_tiling_on_sc=False, needs_layout_passes=False),
        scratch_types=[
            pltpu.VMEM_SHARED((NP,), jnp.float32),
            pltpu.VMEM((PB, 1, B), jnp.int32),
            pltpu.VMEM((PB, 1, B), jnp.int32),
            pltpu.VMEM((3, KS, B), jnp.int32),
            pltpu.VMEM((3, KS, B), jnp.int32),
            pltpu.VMEM((B,), jnp.float32),
            pltpu.VMEM((16,), jnp.int32),
        ] + [pltpu.SemaphoreType.DMA] * 6,
    )(idx2, z1)


def _layer_body(plist, counts, v, z2, seg_out, acc, src_c, dst_c, rows, cnt_v,
                semi0, semi1, semi2, semi3, semi4, semi5,
                semg0, semg1, semg2, sems0, sems1, sems2):
    c = lax.axis_index("c")
    s = lax.axis_index("s")
    tidx = c * NT + s
    semi = (semi0, semi1, semi2, semi3, semi4, semi5)
    semg = (semg0, semg1, semg2)
    sems_ = (sems0, sems1, sems2)

    r0 = s * RT
    pltpu.sync_copy(z2.at[pl.ds(r0, RT)], acc.at[pl.ds(r0, RT)])
    pltpu.sync_copy(counts.at[tidx, 0], cnt_v)
    plsc.subcore_barrier()

    w = jnp.max(cnt_v[pl.ds(0, 16)])
    nb6 = jnp.right_shift((jnp.right_shift(w + 127, 7) + 5) * 2731, 14)
    ntrip = jnp.maximum(nb6, 1)

    for q in range(2):
        pltpu.async_copy(plist.at[0, tidx, q, 0], src_c.at[q, 0], semi[q])
        pltpu.async_copy(plist.at[1, tidx, q, 0], dst_c.at[q, 0], semi[q])

    def outer(t, carry):
        for j in range(NQ):
            i = t * NQ + j
            r = j % NR
            rp = (j - 1) % NR
            qn = (j + 2) % NQ
            qp = (j - 1) % NQ

            @pl.when(i >= 3)
            def _drain_scatter():
                pltpu.make_async_copy(
                    rows.at[r], acc.at[dst_c.at[j, 0]], sems_[r]
                ).wait()

            @pl.when(i + 2 < ntrip * NQ)
            def _prefetch():
                pltpu.async_copy(
                    plist.at[0, tidx, i + 2, 0], src_c.at[qn, 0], semi[qn]
                )
                pltpu.async_copy(
                    plist.at[1, tidx, i + 2, 0], dst_c.at[qn, 0], semi[qn]
                )

            pltpu.make_async_copy(
                plist.at[0, tidx, i, 0], src_c.at[j, 0], semi[j]
            ).wait()
            pltpu.make_async_copy(
                plist.at[1, tidx, i, 0], dst_c.at[j, 0], semi[j]
            ).wait()

            pltpu.async_copy(v.at[src_c.at[j, 0]], rows.at[r], semg[r])

            @pl.when(i >= 1)
            def _advance_prev():
                pltpu.make_async_copy(
                    v.at[src_c.at[qp, 0]], rows.at[rp], semg[rp]
                ).wait()
                pltpu.async_copy(
                    rows.at[rp], acc.at[dst_c.at[qp, 0]], sems_[rp], add=True
                )
        return carry

    lax.fori_loop(0, ntrip, outer, 0)

    pltpu.make_async_copy(
        v.at[src_c.at[NQ - 1, 0]], rows.at[(NQ - 1) % NR], semg[(NQ - 1) % NR]
    ).wait()
    pltpu.async_copy(
        rows.at[(NQ - 1) % NR], acc.at[dst_c.at[NQ - 1, 0]],
        sems_[(NQ - 1) % NR], add=True,
    )
    for slot, p in ((3, 0), (4, 1), (5, 2)):
        pltpu.make_async_copy(rows.at[p], acc.at[dst_c.at[slot, 0]], sems_[p]).wait()

    plsc.subcore_barrier()
    pltpu.sync_copy(
        acc.at[pl.ds(r0, RT)], seg_out.at[pl.ds(c * HALF + r0, RT)]
    )


def _layer_call(plist, counts, v, z2):
    return pl.kernel(
        _layer_body,
        out_type=jax.ShapeDtypeStruct((NP, D), jnp.float32),
        mesh=_sc_mesh(),
        compiler_params=pltpu.CompilerParams(use_tc_tiling_on_sc=False, needs_layout_passes=False),
        scratch_types=[
            pltpu.VMEM_SHARED((HALFP, D), jnp.float32),
            pltpu.VMEM((NQ, 1, B), jnp.int32),
            pltpu.VMEM((NQ, 1, B), jnp.int32),
            pltpu.VMEM((NR, B, D), jnp.float32),
            pltpu.VMEM((16,), jnp.int32),
        ] + [pltpu.SemaphoreType.DMA] * 12,
    )(plist, counts, v, z2)


_RS = 512


def _setup_body(ds_ref, dd_ref, emb_ref, bexp_ref, cexp_ref, v0_ref):
    a = lax.rsqrt(jnp.maximum(ds_ref[...], 1.0))
    b = lax.rsqrt(jnp.maximum(dd_ref[...], 1.0))
    bexp_ref[...] = jnp.broadcast_to(b, (_RS, D))
    cexp_ref[...] = jnp.broadcast_to(a * b, (_RS, D))
    v0_ref[...] = a * emb_ref[...]


def _setup_call(ds_col, dd_col, emb):
    grid = (NP // _RS,)
    return pl.pallas_call(
        _setup_body,
        grid=grid,
        in_specs=[
            pl.BlockSpec((_RS, 1), lambda r: (r, 0)),
            pl.BlockSpec((_RS, 1), lambda r: (r, 0)),
            pl.BlockSpec((_RS, D), lambda r: (r, 0)),
        ],
        out_specs=[
            pl.BlockSpec((_RS, D), lambda r: (r, 0)),
            pl.BlockSpec((_RS, D), lambda r: (r, 0)),
            pl.BlockSpec((_RS, D), lambda r: (r, 0)),
        ],
        out_shape=[
            jax.ShapeDtypeStruct((NP, D), jnp.float32),
            jax.ShapeDtypeStruct((NP, D), jnp.float32),
            jax.ShapeDtypeStruct((NP, D), jnp.float32),
        ],
    )(ds_col, dd_col, emb)


_FR = 6272
_FC = 512
_RB = 392


def _mid_body(s_ref, acc_ref, b_ref, c_ref, accn_ref, vn_ref):
    sv = s_ref[...]
    accn_ref[...] = acc_ref[...] + b_ref[...] * sv
    vn_ref[...] = c_ref[...] * sv


def _mid_call(s_f, acc_f, bexp_f, cexp_f):
    grid = (_FR // _RB,)
    blk = pl.BlockSpec((_RB, _FC), lambda r: (r, 0))
    return pl.pallas_call(
        _mid_body,
        grid=grid,
        in_specs=[blk, blk, blk, blk],
        out_specs=[blk, blk],
        out_shape=[
            jax.ShapeDtypeStruct((_FR, _FC), jnp.float32),
            jax.ShapeDtypeStruct((_FR, _FC), jnp.float32),
        ],
    )(s_f, acc_f, bexp_f, cexp_f)


def _final_body(s_ref, acc_ref, b_ref, out_ref):
    out_ref[...] = (acc_ref[...] + b_ref[...] * s_ref[...]) * 0.25


def _final_call(s_f, acc_f, bexp_f):
    grid = (_FR // _RB,)
    blk = pl.BlockSpec((_RB, _FC), lambda r: (r, 0))
    return pl.pallas_call(
        _final_body,
        grid=grid,
        in_specs=[blk, blk, blk],
        out_specs=blk,
        out_shape=jax.ShapeDtypeStruct((_FR, _FC), jnp.float32),
    )(s_f, acc_f, bexp_f)


def kernel(edge_index, user_table, item_table):
    src = edge_index[0].reshape(NT, E_TILE)
    dst = edge_index[1].reshape(NT, E_TILE)
    pad = N + (jnp.arange(EPS - E_TILE, dtype=jnp.int32) % 128)
    pad = jnp.broadcast_to(pad, (NT, EPS - E_TILE))
    idx2 = jnp.stack(
        [jnp.concatenate([src, pad], axis=1), jnp.concatenate([dst, pad], axis=1)]
    ).reshape(2, NT * NCHS, KS, B)

    emb = jnp.concatenate([user_table, item_table], axis=0)
    emb = jnp.concatenate([emb, jnp.zeros((NP - N, D), jnp.float32)], axis=0)

    z1 = jnp.zeros((NP,), jnp.float32)
    z2 = jnp.zeros((HALF, D), jnp.float32)

    deg2, plist, counts = _part_call(idx2, z1)
    bexp, cexp, v = _setup_call(
        deg2[0].reshape(NP, 1), deg2[1].reshape(NP, 1), emb
    )

    bexp_f = bexp.reshape(_FR, _FC)
    cexp_f = cexp.reshape(_FR, _FC)

    acc_f = emb.reshape(_FR, _FC)
    for _ in range(LAYERS - 1):
        sseg = _layer_call(plist, counts, v, z2)
        acc_f, v_f = _mid_call(sseg.reshape(_FR, _FC), acc_f, bexp_f, cexp_f)
        v = v_f.reshape(NP, D)

    sseg = _layer_call(plist, counts, v, z2)
    out = _final_call(sseg.reshape(_FR, _FC), acc_f, bexp_f)
    return out.reshape(NP, D)[:N]
